# Initial kernel scaffold; baseline (speedup 1.0000x reference)
import jax
import jax.numpy as jnp
from jax.experimental import pallas as pl


def kernel(pos_enc, x):
    raise NotImplementedError("write your pallas kernel here")


# speedup vs baseline: 2.3031x; 2.3031x over previous
import functools

import jax
import jax.numpy as jnp
from jax import lax
from jax.experimental import pallas as pl
from jax.experimental.pallas import tpu as pltpu
from jax.experimental.pallas import tpu_sc as plsc

D_MODEL = 1024
BATCH = 4
SEQ = 8192
B_TOTAL = BATCH * SEQ
NUM_CORES = 2
NUM_SUBCORES = 16
NW = NUM_CORES * NUM_SUBCORES
B_PER_W = B_TOTAL // NW
CHUNK = 32
N_CHUNKS = B_PER_W // CHUNK

_mesh = plsc.VectorSubcoreMesh(core_axis_name="c", subcore_axis_name="s")


@functools.partial(
    pl.kernel,
    mesh=_mesh,
    out_type=jax.ShapeDtypeStruct((B_TOTAL, D_MODEL), jnp.float32),
    scratch_types=[
        pltpu.VMEM((N_CHUNKS, CHUNK), jnp.int32),
        pltpu.VMEM((CHUNK, D_MODEL), jnp.float32),
        pltpu.VMEM((CHUNK, D_MODEL), jnp.float32),
        pltpu.SemaphoreType.DMA,
        pltpu.SemaphoreType.DMA,
    ],
)
def _gather_rows(table_hbm, idx_hbm, out_hbm, idx_v, buf0, buf1, gsem, osem):
    wid = lax.axis_index("s") * NUM_CORES + lax.axis_index("c")
    base = wid * B_PER_W
    pltpu.sync_copy(idx_hbm.at[wid], idx_v)

    def body(i, carry):
        g0 = 2 * i
        g1 = g0 + 1
        in0 = pltpu.async_copy(table_hbm.at[idx_v.at[g0]], buf0, gsem)
        in0.wait()

        @pl.when(i > 0)
        def _():
            pltpu.make_async_copy(
                buf1, out_hbm.at[pl.ds(base, CHUNK)], osem
            ).wait()

        in1 = pltpu.async_copy(table_hbm.at[idx_v.at[g1]], buf1, gsem)
        out0 = pltpu.async_copy(
            buf0, out_hbm.at[pl.ds(base + g0 * CHUNK, CHUNK)], osem
        )
        in1.wait()
        out0.wait()
        pltpu.async_copy(
            buf1, out_hbm.at[pl.ds(base + g1 * CHUNK, CHUNK)], osem
        )
        return carry

    lax.fori_loop(0, N_CHUNKS // 2, body, 0)
    pltpu.make_async_copy(buf1, out_hbm.at[pl.ds(base, CHUNK)], osem).wait()


def kernel(pos_enc, x):
    idx = jnp.asarray(x, jnp.int32).reshape(NW, N_CHUNKS, CHUNK)
    out = _gather_rows(jnp.asarray(pos_enc, jnp.float32), idx)
    return out.reshape(BATCH, SEQ, D_MODEL)


# speedup vs baseline: 2.3871x; 1.0365x over previous
import functools

import jax
import jax.numpy as jnp
from jax import lax
from jax.experimental import pallas as pl
from jax.experimental.pallas import tpu as pltpu
from jax.experimental.pallas import tpu_sc as plsc

D_MODEL = 1024
BATCH = 4
SEQ = 8192
B_TOTAL = BATCH * SEQ
NUM_CORES = 2
NUM_SUBCORES = 16
NW = NUM_CORES * NUM_SUBCORES
B_PER_W = B_TOTAL // NW
CHUNK = 16
N_CHUNKS = B_PER_W // CHUNK
NBUF = 4

_mesh = plsc.VectorSubcoreMesh(core_axis_name="c", subcore_axis_name="s")


@functools.partial(
    pl.kernel,
    mesh=_mesh,
    out_type=jax.ShapeDtypeStruct((B_TOTAL, D_MODEL), jnp.float32),
    scratch_types=[
        pltpu.VMEM((N_CHUNKS, CHUNK), jnp.int32),
        pltpu.VMEM((NBUF, CHUNK, D_MODEL), jnp.float32),
        pltpu.SemaphoreType.DMA((NBUF,)),
        pltpu.SemaphoreType.DMA((NBUF,)),
    ],
)
def _gather_rows(table_hbm, idx_hbm, out_hbm, idx_v, bufs, gsems, osems):
    wid = lax.axis_index("s") * NUM_CORES + lax.axis_index("c")
    base = wid * B_PER_W
    pltpu.sync_copy(idx_hbm.at[wid], idx_v)

    for b in range(NBUF - 1):
        pltpu.async_copy(table_hbm.at[idx_v.at[b]], bufs.at[b], gsems.at[b])

    def body(i, carry):
        for b in range(NBUF):
            g = NBUF * i + b
            prev = (b + NBUF - 1) % NBUF
            pltpu.make_async_copy(
                table_hbm.at[idx_v.at[0]], bufs.at[b], gsems.at[b]
            ).wait()
            @pl.when(g > 0)
            def _():
                pltpu.make_async_copy(
                    bufs.at[prev],
                    out_hbm.at[pl.ds(base, CHUNK)],
                    osems.at[prev],
                ).wait()

            @pl.when(g + NBUF - 1 < N_CHUNKS)
            def _():
                pltpu.async_copy(
                    table_hbm.at[idx_v.at[g + NBUF - 1]],
                    bufs.at[prev],
                    gsems.at[prev],
                )

            pltpu.async_copy(
                bufs.at[b],
                out_hbm.at[pl.ds(base + g * CHUNK, CHUNK)],
                osems.at[b],
            )
        return carry

    lax.fori_loop(0, N_CHUNKS // NBUF, body, 0)
    pltpu.make_async_copy(
        bufs.at[NBUF - 1], out_hbm.at[pl.ds(base, CHUNK)], osems.at[NBUF - 1]
    ).wait()


def kernel(pos_enc, x):
    idx = jnp.asarray(x, jnp.int32).reshape(NW, N_CHUNKS, CHUNK)
    out = _gather_rows(jnp.asarray(pos_enc, jnp.float32), idx)
    return out.reshape(BATCH, SEQ, D_MODEL)
